# baseline (speedup 1.0000x reference)
import functools

import jax
import jax.numpy as jnp
import numpy as np
from jax import lax
from jax.experimental import pallas as pl
from jax.experimental.pallas import tpu as pltpu
from jax.experimental.pallas import tpu_sc as plsc

CUTOFF = 10.0

_NC = 2
_NS = 16
_NW = _NC * _NS
_LANES = 16


_SSP_COEF = (3.5569863712225924e-06, 0.12497252665845734, -0.005172411501547311,
             0.0003282103940751628, -2.088206334681123e-05,
             1.1289928774409745e-06, -4.6605211875865655e-08,
             1.3612389304211794e-09, -2.609402312738333e-11,
             2.921410155424878e-13, -1.4413872274671075e-15)


_COS_COEF = (0.9999999922898464, -0.49999991770959556, 0.04166652433757511,
             -0.0013887970265677564, 2.4773420813744614e-05,
             -2.7113337726357255e-07, 1.7368996060426488e-09)


def _ssp(t):
    a = jnp.abs(t)
    u = jnp.minimum(t * t, 36.0)
    g = jnp.float32(_SSP_COEF[-1])
    for coef in _SSP_COEF[-2::-1]:
        g = g * u + jnp.float32(coef)
    return 0.5 * t + g + 0.5 * jnp.maximum(a - 6.0, 0.0)


def _h_body(x_ref, w1_ref, o_ref):
    o_ref[...] = lax.dot_general(
        x_ref[...], w1_ref[...], (((1,), (1,)), ((), ())),
        preferred_element_type=jnp.float32)


def _compute_h(x, W1):
    N, H = x.shape
    return pl.pallas_call(
        _h_body,
        out_shape=jax.ShapeDtypeStruct((N, W1.shape[0]), jnp.float32),
    )(x, W1)


def _w_body(ea_ref, el_ref, w1_ref, b1_ref, w2_ref, b2_ref, o_ref):
    t = lax.dot_general(ea_ref[...], w1_ref[...], (((1,), (1,)), ((), ())),
                        preferred_element_type=jnp.float32)
    t = _ssp(t + b1_ref[...])
    t = lax.dot_general(t, w2_ref[...], (((1,), (1,)), ((), ())),
                        preferred_element_type=jnp.float32)
    t = t + b2_ref[...]
    el = el_ref[...]
    y2 = jnp.square(el * (np.pi / CUTOFF))
    cosv = jnp.float32(_COS_COEF[-1])
    for coef in _COS_COEF[-2::-1]:
        cosv = cosv * y2 + jnp.float32(coef)
    c = 0.5 * (cosv + 1.0)
    c = c * (el <= CUTOFF).astype(jnp.float32) * (el >= 0.0).astype(jnp.float32)
    o_ref[...] = t * c


def _compute_w(edge_attr, edge_length, dn_w1, dn_b1, dn_w2, dn_b2):
    E, G = edge_attr.shape
    F = dn_w1.shape[0]
    BE = 3200
    grid = E // BE
    el2 = edge_length.reshape(E, 1)
    return pl.pallas_call(
        _w_body,
        grid=(grid,),
        in_specs=[
            pl.BlockSpec((BE, G), lambda i: (i, 0)),
            pl.BlockSpec((BE, 1), lambda i: (i, 0)),
            pl.BlockSpec((F, G), lambda i: (0, 0)),
            pl.BlockSpec((1, F), lambda i: (0, 0)),
            pl.BlockSpec((F, F), lambda i: (0, 0)),
            pl.BlockSpec((1, F), lambda i: (0, 0)),
        ],
        out_specs=pl.BlockSpec((BE, F), lambda i: (i, 0)),
        out_shape=jax.ShapeDtypeStruct((E, F), jnp.float32),
    )(edge_attr, el2, dn_w1, dn_b1.reshape(1, F), dn_w2, dn_b2.reshape(1, F))


def _sc_gms(h, w, src, dst):
    N, H = h.shape
    E = w.shape[0]
    per_w = E // _NW
    CH = 80
    n_chunks = per_w // CH
    assert per_w % CH == 0 and H == 128
    ZR = 16
    npad = -(-N // (_NS * ZR)) * _NS * ZR
    rows_per_tile = npad // _NS
    n_bounce = rows_per_tile // ZR
    HV = H // _LANES

    src3 = src.reshape(_NW, n_chunks, CH)
    dst3 = dst.reshape(_NW, n_chunks, CH)

    def body(h_hbm, w_hbm, src_hbm, dst_hbm, out_hbm,
             src_v, dst_v, hbuf, wbuf, bb_v, agg_sh):
        c = lax.axis_index("c")
        s = lax.axis_index("s")
        wid = s * _NC + c

        zvec = jnp.zeros((_LANES,), jnp.float32)

        def zrow(r, _):
            for k in range(HV):
                bb_v[r, pl.ds(k * _LANES, _LANES)] = zvec
            return 0
        lax.fori_loop(0, ZR, zrow, 0)

        def zcopy(j, _):
            pltpu.sync_copy(bb_v, agg_sh.at[pl.ds(s * rows_per_tile + j * ZR, ZR)])
            return 0
        lax.fori_loop(0, n_bounce, zcopy, 0)

        plsc.subcore_barrier()

        base0 = wid * per_w

        def chunk(i, _):
            pltpu.sync_copy(src_hbm.at[wid, i], src_v)
            pltpu.sync_copy(dst_hbm.at[wid, i], dst_v)
            pltpu.sync_copy(h_hbm.at[src_v], hbuf)
            pltpu.sync_copy(w_hbm.at[pl.ds(base0 + i * CH, CH)], wbuf)

            def mrow(r, _):
                for k in range(HV):
                    sl = pl.ds(k * _LANES, _LANES)
                    wbuf[r, sl] = wbuf[r, sl] * hbuf[r, sl]
                return 0
            lax.fori_loop(0, CH, mrow, 0)
            pltpu.sync_copy(wbuf, agg_sh.at[dst_v], add=True)
            return 0
        lax.fori_loop(0, n_chunks, chunk, 0)
        plsc.subcore_barrier()

        def ocopy(j, _):
            r0 = s * rows_per_tile + j * ZR
            pltpu.sync_copy(agg_sh.at[pl.ds(r0, ZR)], bb_v)
            pltpu.sync_copy(bb_v, out_hbm.at[c, pl.ds(r0, ZR)])
            return 0
        lax.fori_loop(0, n_bounce, ocopy, 0)

    mesh = plsc.VectorSubcoreMesh(core_axis_name="c", subcore_axis_name="s")
    return pl.kernel(
        body,
        out_type=jax.ShapeDtypeStruct((_NC, npad, H), jnp.float32),
        mesh=mesh,
        scratch_types=[
            pltpu.VMEM((CH,), jnp.int32),
            pltpu.VMEM((CH,), jnp.int32),
            pltpu.VMEM((CH, H), jnp.float32),
            pltpu.VMEM((CH, H), jnp.float32),
            pltpu.VMEM((ZR, H), jnp.float32),
            pltpu.VMEM_SHARED((npad, H), jnp.float32),
        ],
    )(h, w, src3, dst3)


def _out_body(p_ref, w2_ref, b2_ref, lw_ref, lb_ref, o_ref):
    agg = p_ref[0] + p_ref[1]
    t = lax.dot_general(agg, w2_ref[...], (((1,), (1,)), ((), ())),
                        preferred_element_type=jnp.float32)
    t = _ssp(t + b2_ref[...])
    o_ref[...] = lax.dot_general(t, lw_ref[...], (((1,), (1,)), ((), ())),
                                 preferred_element_type=jnp.float32) + lb_ref[...]


def _compute_out(partials, N, W2, b2, lin_w, lin_b):
    F = partials.shape[2]
    H = W2.shape[0]
    BN = 1000
    grid = N // BN
    return pl.pallas_call(
        _out_body,
        grid=(grid,),
        in_specs=[
            pl.BlockSpec((_NC, BN, F), lambda i: (0, i, 0)),
            pl.BlockSpec((H, F), lambda i: (0, 0)),
            pl.BlockSpec((1, H), lambda i: (0, 0)),
            pl.BlockSpec((H, H), lambda i: (0, 0)),
            pl.BlockSpec((1, H), lambda i: (0, 0)),
        ],
        out_specs=pl.BlockSpec((BN, H), lambda i: (i, 0)),
        out_shape=jax.ShapeDtypeStruct((N, H), jnp.float32),
    )(partials, W2, b2.reshape(1, H), lin_w, lin_b.reshape(1, H))


def kernel(x, edge_index, edge_length, edge_attr,
           W1, dn_w1, dn_b1, dn_w2, dn_b2, W2, b2, lin_w, lin_b):
    h = _compute_h(x, W1)
    w = _compute_w(edge_attr, edge_length, dn_w1, dn_b1, dn_w2, dn_b2)
    src = edge_index[0]
    dst = edge_index[1]
    partials = _sc_gms(h, w, src, dst)
    return _compute_out(partials, x.shape[0], W2, b2, lin_w, lin_b)


# speedup vs baseline: 1.2899x; 1.2899x over previous
import functools

import jax
import jax.numpy as jnp
import numpy as np
from jax import lax
from jax.experimental import pallas as pl
from jax.experimental.pallas import tpu as pltpu
from jax.experimental.pallas import tpu_sc as plsc

CUTOFF = 10.0

_NC = 2
_NS = 16
_NW = _NC * _NS
_LANES = 16


_SSP_COEF = (3.5569863712225924e-06, 0.12497252665845734, -0.005172411501547311,
             0.0003282103940751628, -2.088206334681123e-05,
             1.1289928774409745e-06, -4.6605211875865655e-08,
             1.3612389304211794e-09, -2.609402312738333e-11,
             2.921410155424878e-13, -1.4413872274671075e-15)


_COS_COEF = (0.9999999922898464, -0.49999991770959556, 0.04166652433757511,
             -0.0013887970265677564, 2.4773420813744614e-05,
             -2.7113337726357255e-07, 1.7368996060426488e-09)


def _ssp(t):
    a = jnp.abs(t)
    u = jnp.minimum(t * t, 36.0)
    g = jnp.float32(_SSP_COEF[-1])
    for coef in _SSP_COEF[-2::-1]:
        g = g * u + jnp.float32(coef)
    return 0.5 * t + g + 0.5 * jnp.maximum(a - 6.0, 0.0)


def _h_body(x_ref, w1_ref, o_ref):
    o_ref[...] = lax.dot_general(
        x_ref[...], w1_ref[...], (((1,), (1,)), ((), ())),
        preferred_element_type=jnp.float32)


def _compute_h(x, W1):
    N, H = x.shape
    return pl.pallas_call(
        _h_body,
        out_shape=jax.ShapeDtypeStruct((N, W1.shape[0]), jnp.float32),
    )(x, W1)


def _w_body(ea_ref, el_ref, w1_ref, b1_ref, w2_ref, b2_ref, o_ref):
    t = lax.dot_general(ea_ref[...], w1_ref[...], (((0,), (1,)), ((), ())),
                        preferred_element_type=jnp.float32)
    t = _ssp(t + b1_ref[...])
    t = lax.dot_general(t, w2_ref[...], (((1,), (1,)), ((), ())),
                        preferred_element_type=jnp.float32)
    t = t + b2_ref[...]
    BE, F = t.shape
    el = el_ref[...]
    y2 = jnp.square(el * (np.pi / CUTOFF))
    cosv = jnp.float32(_COS_COEF[-1])
    for coef in _COS_COEF[-2::-1]:
        cosv = cosv * y2 + jnp.float32(coef)
    c = 0.5 * (cosv + 1.0)
    c = c * (el <= CUTOFF).astype(jnp.float32) * (el >= 0.0).astype(jnp.float32)
    o_ref[...] = t * lax.transpose(c, (1, 0))


def _compute_w(edge_attr, edge_length, dn_w1, dn_b1, dn_w2, dn_b2):
    E, G = edge_attr.shape
    F = dn_w1.shape[0]
    BE = 3200
    grid = E // BE
    el_t = edge_length.reshape(1, E)
    ea_t = edge_attr.T
    return pl.pallas_call(
        _w_body,
        grid=(grid,),
        in_specs=[
            pl.BlockSpec((G, BE), lambda i: (0, i)),
            pl.BlockSpec((1, BE), lambda i: (0, i)),
            pl.BlockSpec((F, G), lambda i: (0, 0)),
            pl.BlockSpec((1, F), lambda i: (0, 0)),
            pl.BlockSpec((F, F), lambda i: (0, 0)),
            pl.BlockSpec((1, F), lambda i: (0, 0)),
        ],
        out_specs=pl.BlockSpec((BE, F), lambda i: (i, 0)),
        out_shape=jax.ShapeDtypeStruct((E, F), jnp.float32),
    )(ea_t, el_t, dn_w1, dn_b1.reshape(1, F), dn_w2, dn_b2.reshape(1, F))


def _sc_gms(h, w, src, dst):
    N, H = h.shape
    E = w.shape[0]
    per_w = E // _NW
    CH = 80
    n_chunks = per_w // CH
    assert per_w % CH == 0 and H == 128
    ZR = 16
    npad = -(-N // (_NS * ZR)) * _NS * ZR
    rows_per_tile = npad // _NS
    n_bounce = rows_per_tile // ZR
    HV = H // _LANES

    src3 = src.reshape(_NW, n_chunks, CH)
    dst3 = dst.reshape(_NW, n_chunks, CH)

    def body(h_hbm, w_hbm, src_hbm, dst_hbm, out_hbm,
             src_v, dst_v, hbuf, wbuf, bb_v, agg_sh):
        c = lax.axis_index("c")
        s = lax.axis_index("s")
        wid = s * _NC + c

        zvec = jnp.zeros((_LANES,), jnp.float32)

        def zrow(r, _):
            for k in range(HV):
                bb_v[r, pl.ds(k * _LANES, _LANES)] = zvec
            return 0
        lax.fori_loop(0, ZR, zrow, 0)

        def zcopy(j, _):
            pltpu.sync_copy(bb_v, agg_sh.at[pl.ds(s * rows_per_tile + j * ZR, ZR)])
            return 0
        lax.fori_loop(0, n_bounce, zcopy, 0)

        plsc.subcore_barrier()

        base0 = wid * per_w

        def chunk(i, _):
            pltpu.sync_copy(src_hbm.at[wid, i], src_v)
            pltpu.sync_copy(dst_hbm.at[wid, i], dst_v)
            pltpu.sync_copy(h_hbm.at[src_v], hbuf)
            pltpu.sync_copy(w_hbm.at[pl.ds(base0 + i * CH, CH)], wbuf)

            def mrow(r, _):
                for k in range(HV):
                    sl = pl.ds(k * _LANES, _LANES)
                    wbuf[r, sl] = wbuf[r, sl] * hbuf[r, sl]
                return 0
            lax.fori_loop(0, CH, mrow, 0)
            pltpu.sync_copy(wbuf, agg_sh.at[dst_v], add=True)
            return 0
        lax.fori_loop(0, n_chunks, chunk, 0)
        plsc.subcore_barrier()

        def ocopy(j, _):
            r0 = s * rows_per_tile + j * ZR
            pltpu.sync_copy(agg_sh.at[pl.ds(r0, ZR)], bb_v)
            pltpu.sync_copy(bb_v, out_hbm.at[c, pl.ds(r0, ZR)])
            return 0
        lax.fori_loop(0, n_bounce, ocopy, 0)

    mesh = plsc.VectorSubcoreMesh(core_axis_name="c", subcore_axis_name="s")
    return pl.kernel(
        body,
        out_type=jax.ShapeDtypeStruct((_NC, npad, H), jnp.float32),
        mesh=mesh,
        scratch_types=[
            pltpu.VMEM((CH,), jnp.int32),
            pltpu.VMEM((CH,), jnp.int32),
            pltpu.VMEM((CH, H), jnp.float32),
            pltpu.VMEM((CH, H), jnp.float32),
            pltpu.VMEM((ZR, H), jnp.float32),
            pltpu.VMEM_SHARED((npad, H), jnp.float32),
        ],
    )(h, w, src3, dst3)


def _out_body(p_ref, w2_ref, b2_ref, lw_ref, lb_ref, o_ref):
    agg = p_ref[0] + p_ref[1]
    t = lax.dot_general(agg, w2_ref[...], (((1,), (1,)), ((), ())),
                        preferred_element_type=jnp.float32)
    t = _ssp(t + b2_ref[...])
    o_ref[...] = lax.dot_general(t, lw_ref[...], (((1,), (1,)), ((), ())),
                                 preferred_element_type=jnp.float32) + lb_ref[...]


def _compute_out(partials, N, W2, b2, lin_w, lin_b):
    F = partials.shape[2]
    H = W2.shape[0]
    BN = 1000
    grid = N // BN
    return pl.pallas_call(
        _out_body,
        grid=(grid,),
        in_specs=[
            pl.BlockSpec((_NC, BN, F), lambda i: (0, i, 0)),
            pl.BlockSpec((H, F), lambda i: (0, 0)),
            pl.BlockSpec((1, H), lambda i: (0, 0)),
            pl.BlockSpec((H, H), lambda i: (0, 0)),
            pl.BlockSpec((1, H), lambda i: (0, 0)),
        ],
        out_specs=pl.BlockSpec((BN, H), lambda i: (i, 0)),
        out_shape=jax.ShapeDtypeStruct((N, H), jnp.float32),
    )(partials, W2, b2.reshape(1, H), lin_w, lin_b.reshape(1, H))


def kernel(x, edge_index, edge_length, edge_attr,
           W1, dn_w1, dn_b1, dn_w2, dn_b2, W2, b2, lin_w, lin_b):
    h = _compute_h(x, W1)
    w = _compute_w(edge_attr, edge_length, dn_w1, dn_b1, dn_w2, dn_b2)
    src = edge_index[0]
    dst = edge_index[1]
    partials = _sc_gms(h, w, src, dst)
    return _compute_out(partials, x.shape[0], W2, b2, lin_w, lin_b)


# speedup vs baseline: 1.4422x; 1.1180x over previous
import functools

import jax
import jax.numpy as jnp
import numpy as np
from jax import lax
from jax.experimental import pallas as pl
from jax.experimental.pallas import tpu as pltpu
from jax.experimental.pallas import tpu_sc as plsc

CUTOFF = 10.0

_NC = 2
_NS = 16
_NW = _NC * _NS
_LANES = 16


_SSP_COEF = (3.5569863712225924e-06, 0.12497252665845734, -0.005172411501547311,
             0.0003282103940751628, -2.088206334681123e-05,
             1.1289928774409745e-06, -4.6605211875865655e-08,
             1.3612389304211794e-09, -2.609402312738333e-11,
             2.921410155424878e-13, -1.4413872274671075e-15)


_COS_COEF = (0.9999999922898464, -0.49999991770959556, 0.04166652433757511,
             -0.0013887970265677564, 2.4773420813744614e-05,
             -2.7113337726357255e-07, 1.7368996060426488e-09)


def _ssp(t):
    a = jnp.abs(t)
    u = jnp.minimum(t * t, 36.0)
    g = jnp.float32(_SSP_COEF[-1])
    for coef in _SSP_COEF[-2::-1]:
        g = g * u + jnp.float32(coef)
    return 0.5 * t + g + 0.5 * jnp.maximum(a - 6.0, 0.0)


def _h_body(x_ref, w1_ref, o_ref):
    o_ref[...] = lax.dot_general(
        x_ref[...], w1_ref[...], (((1,), (1,)), ((), ())),
        preferred_element_type=jnp.float32)


def _compute_h(x, W1):
    N, H = x.shape
    return pl.pallas_call(
        _h_body,
        out_shape=jax.ShapeDtypeStruct((N, W1.shape[0]), jnp.float32),
    )(x, W1)


def _w_body(ea_ref, el_ref, w1_ref, b1_ref, w2_ref, b2_ref, o_ref):
    t = lax.dot_general(ea_ref[...], w1_ref[...], (((0,), (1,)), ((), ())),
                        preferred_element_type=jnp.float32)
    t = _ssp(t + b1_ref[...])
    t = lax.dot_general(t, w2_ref[...], (((1,), (1,)), ((), ())),
                        preferred_element_type=jnp.float32)
    t = t + b2_ref[...]
    BE, F = t.shape
    el = el_ref[...]
    y2 = jnp.square(el * (np.pi / CUTOFF))
    cosv = jnp.float32(_COS_COEF[-1])
    for coef in _COS_COEF[-2::-1]:
        cosv = cosv * y2 + jnp.float32(coef)
    c = 0.5 * (cosv + 1.0)
    c = c * (el <= CUTOFF).astype(jnp.float32) * (el >= 0.0).astype(jnp.float32)
    o_ref[...] = t * lax.transpose(c, (1, 0))


def _compute_w(ea_t, el_t, dn_w1, dn_b1, dn_w2, dn_b2, e0, ep):
    G = ea_t.shape[0]
    F = dn_w1.shape[0]
    BE = 3200
    grid = ep // BE
    blk0 = e0 // BE
    return pl.pallas_call(
        _w_body,
        grid=(grid,),
        in_specs=[
            pl.BlockSpec((G, BE), lambda i: (0, i + blk0)),
            pl.BlockSpec((1, BE), lambda i: (0, i + blk0)),
            pl.BlockSpec((F, G), lambda i: (0, 0)),
            pl.BlockSpec((1, F), lambda i: (0, 0)),
            pl.BlockSpec((F, F), lambda i: (0, 0)),
            pl.BlockSpec((1, F), lambda i: (0, 0)),
        ],
        out_specs=pl.BlockSpec((BE, F), lambda i: (i, 0)),
        out_shape=jax.ShapeDtypeStruct((ep, F), jnp.float32),
    )(ea_t, el_t, dn_w1, dn_b1.reshape(1, F), dn_w2, dn_b2.reshape(1, F))


def _sc_gms(h, w, src3, dst3):
    N, H = h.shape
    E = w.shape[0]
    per_w = E // _NW
    CH = src3.shape[2]
    n_chunks = src3.shape[1]
    assert per_w == CH * n_chunks and H == 128
    ZR = 16
    npad = -(-N // (_NS * ZR)) * _NS * ZR
    rows_per_tile = npad // _NS
    n_bounce = rows_per_tile // ZR
    HV = H // _LANES

    def body(h_hbm, w_hbm, src_hbm, dst_hbm, out_hbm,
             src_v, dst_v, hbuf, wbuf, bb_v, agg_sh):
        c = lax.axis_index("c")
        s = lax.axis_index("s")
        wid = s * _NC + c

        zvec = jnp.zeros((_LANES,), jnp.float32)

        def zrow(r, _):
            for k in range(HV):
                bb_v[r, pl.ds(k * _LANES, _LANES)] = zvec
            return 0
        lax.fori_loop(0, ZR, zrow, 0)

        def zcopy(j, _):
            pltpu.sync_copy(bb_v, agg_sh.at[pl.ds(s * rows_per_tile + j * ZR, ZR)])
            return 0
        lax.fori_loop(0, n_bounce, zcopy, 0)

        plsc.subcore_barrier()

        base0 = wid * per_w

        def chunk(i, _):
            pltpu.sync_copy(src_hbm.at[wid, i], src_v)
            pltpu.sync_copy(dst_hbm.at[wid, i], dst_v)
            pltpu.sync_copy(h_hbm.at[src_v], hbuf)
            pltpu.sync_copy(w_hbm.at[pl.ds(base0 + i * CH, CH)], wbuf)

            def mrow(r, _):
                for k in range(HV):
                    sl = pl.ds(k * _LANES, _LANES)
                    wbuf[r, sl] = wbuf[r, sl] * hbuf[r, sl]
                return 0
            lax.fori_loop(0, CH, mrow, 0)
            pltpu.sync_copy(wbuf, agg_sh.at[dst_v], add=True)
            return 0
        lax.fori_loop(0, n_chunks, chunk, 0)
        plsc.subcore_barrier()

        def ocopy(j, _):
            r0 = s * rows_per_tile + j * ZR
            pltpu.sync_copy(agg_sh.at[pl.ds(r0, ZR)], bb_v)
            pltpu.sync_copy(bb_v, out_hbm.at[c, pl.ds(r0, ZR)])
            return 0
        lax.fori_loop(0, n_bounce, ocopy, 0)

    mesh = plsc.VectorSubcoreMesh(core_axis_name="c", subcore_axis_name="s")
    return pl.kernel(
        body,
        out_type=jax.ShapeDtypeStruct((_NC, npad, H), jnp.float32),
        mesh=mesh,
        scratch_types=[
            pltpu.VMEM((CH,), jnp.int32),
            pltpu.VMEM((CH,), jnp.int32),
            pltpu.VMEM((CH, H), jnp.float32),
            pltpu.VMEM((CH, H), jnp.float32),
            pltpu.VMEM((ZR, H), jnp.float32),
            pltpu.VMEM_SHARED((npad, H), jnp.float32),
        ],
    )(h, w, src3, dst3)


def _out_body(p0_ref, p1_ref, p2_ref, w2_ref, b2_ref, lw_ref, lb_ref, o_ref):
    agg = (p0_ref[0] + p0_ref[1] + p1_ref[0] + p1_ref[1]
           + p2_ref[0] + p2_ref[1])
    t = lax.dot_general(agg, w2_ref[...], (((1,), (1,)), ((), ())),
                        preferred_element_type=jnp.float32)
    t = _ssp(t + b2_ref[...])
    o_ref[...] = lax.dot_general(t, lw_ref[...], (((1,), (1,)), ((), ())),
                                 preferred_element_type=jnp.float32) + lb_ref[...]


def _compute_out(partials, N, W2, b2, lin_w, lin_b):
    F = partials[0].shape[2]
    H = W2.shape[0]
    BN = 1000
    grid = N // BN
    pspec = pl.BlockSpec((_NC, BN, F), lambda i: (0, i, 0))
    return pl.pallas_call(
        _out_body,
        grid=(grid,),
        in_specs=[
            pspec, pspec, pspec,
            pl.BlockSpec((H, F), lambda i: (0, 0)),
            pl.BlockSpec((1, H), lambda i: (0, 0)),
            pl.BlockSpec((H, H), lambda i: (0, 0)),
            pl.BlockSpec((1, H), lambda i: (0, 0)),
        ],
        out_specs=pl.BlockSpec((BN, H), lambda i: (i, 0)),
        out_shape=jax.ShapeDtypeStruct((N, H), jnp.float32),
    )(*partials, W2, b2.reshape(1, H), lin_w, lin_b.reshape(1, H))


def kernel(x, edge_index, edge_length, edge_attr,
           W1, dn_w1, dn_b1, dn_w2, dn_b2, W2, b2, lin_w, lin_b):
    E = edge_attr.shape[0]
    h = _compute_h(x, W1)
    el_t = edge_length.reshape(1, E)
    ea_t = edge_attr.T
    src = edge_index[0]
    dst = edge_index[1]
    CH = 80
    bounds = (0, 64000, 192000, E)
    partials = []
    for k in range(3):
        e0, e1 = bounds[k], bounds[k + 1]
        ep = e1 - e0
        w_p = _compute_w(ea_t, el_t, dn_w1, dn_b1, dn_w2, dn_b2, e0, ep)
        nck = ep // (_NW * CH)
        src3 = src[e0:e1].reshape(_NW, nck, CH)
        dst3 = dst[e0:e1].reshape(_NW, nck, CH)
        partials.append(_sc_gms(h, w_p, src3, dst3))
    return _compute_out(partials, x.shape[0], W2, b2, lin_w, lin_b)
